# Initial kernel scaffold; baseline (speedup 1.0000x reference)
import jax
import jax.numpy as jnp
from jax.experimental import pallas as pl


def kernel(features, edge_index, W1, b1, W2, b2):
    raise NotImplementedError("write your pallas kernel here")


# speedup vs baseline: 5.6500x; 5.6500x over previous
import dataclasses
import functools

import jax
import jax.numpy as jnp
from jax import lax
from jax.experimental import pallas as pl
from jax.experimental.pallas import tpu as pltpu
from jax.experimental.pallas import tpu_sc as plsc

N = 10000
E = 320000
D = 128
NC = 2
NS = 16
EC = E // NC
EW = EC // NS
W = 80
NWIN = EW // W
RCH = 640
NP = NS * RCH

_mesh = plsc.VectorSubcoreMesh(core_axis_name="c", subcore_axis_name="s")

_sc_params = pltpu.CompilerParams()
if "needs_layout_passes" in pltpu.CompilerParams.__dataclass_fields__:
    _sc_params = dataclasses.replace(_sc_params, needs_layout_passes=False)


def _degrees(src, dst):
    @functools.partial(
        pl.kernel,
        out_type=jax.ShapeDtypeStruct((NC, 2, NP), jnp.float32),
        mesh=_mesh,
        compiler_params=_sc_params,
        scratch_types=[
            pltpu.VMEM_SHARED((2, NS, NP), jnp.float32),
            pltpu.VMEM((NP,), jnp.float32),
            pltpu.VMEM((NP,), jnp.float32),
            pltpu.VMEM((EW,), jnp.int32),
            pltpu.VMEM((NS, RCH), jnp.float32),
            pltpu.VMEM((RCH,), jnp.float32),
        ],
    )
    def k(src_hbm, dst_hbm, deg_hbm, stage_sh, hs, hd, ebuf, red, outv):
        c = lax.axis_index("c")
        s = lax.axis_index("s")
        ones = jnp.ones((16,), jnp.float32)
        zeros = jnp.zeros((16,), jnp.float32)

        @pl.loop(0, NP, step=16)
        def _(i):
            hs[pl.ds(i, 16)] = zeros
            hd[pl.ds(i, 16)] = zeros

        base = (c * NS + s) * EW
        pltpu.sync_copy(src_hbm.at[pl.ds(base, EW)], ebuf)

        @pl.loop(0, EW, step=16)
        def _(e):
            plsc.addupdate_scatter(hs, [ebuf[pl.ds(e, 16)]], ones)

        pltpu.sync_copy(dst_hbm.at[pl.ds(base, EW)], ebuf)

        @pl.loop(0, EW, step=16)
        def _(e):
            plsc.addupdate_scatter(hd, [ebuf[pl.ds(e, 16)]], ones)

        pltpu.sync_copy(hs, stage_sh.at[0, s])
        pltpu.sync_copy(hd, stage_sh.at[1, s])
        plsc.subcore_barrier()

        el0 = s * RCH
        for which in range(2):
            pltpu.sync_copy(stage_sh.at[which, :, pl.ds(el0, RCH)], red)

            @pl.loop(0, RCH, step=16)
            def _(i):
                acc = red[0, pl.ds(i, 16)]
                for t in range(1, NS):
                    acc = acc + red[t, pl.ds(i, 16)]
                outv[pl.ds(i, 16)] = acc

            pltpu.sync_copy(outv, deg_hbm.at[c, which, pl.ds(el0, RCH)])

    return k(src, dst)


def _edge_pass(h, src, dst):
    @functools.partial(
        pl.kernel,
        out_type=jax.ShapeDtypeStruct((NC, N, D), jnp.float32),
        mesh=_mesh,
        scratch_types=[
            pltpu.VMEM_SHARED((N, D), jnp.float32),
            pltpu.VMEM((128, D), jnp.float32),
            pltpu.VMEM((W,), jnp.int32),
            pltpu.VMEM((W,), jnp.int32),
            pltpu.VMEM((W, D), jnp.float32),
            pltpu.SemaphoreType.DMA,
        ],
    )
    def k(h_hbm, src_hbm, dst_hbm, out_hbm, acc_sh, zbuf, sidx, didx, rows, sem):
        c = lax.axis_index("c")
        s = lax.axis_index("s")
        zeros = jnp.zeros((16,), jnp.float32)

        @pl.loop(0, 128)
        def _(r):
            @pl.loop(0, D, step=16)
            def _(col):
                zbuf[r, pl.ds(col, 16)] = zeros

        row0 = jnp.minimum(s * RCH, N - RCH)
        for j in range(RCH // 128):
            pltpu.sync_copy(zbuf, acc_sh.at[pl.ds(row0 + j * 128, 128)])
        plsc.subcore_barrier()

        base = (c * NS + s) * EW

        @pl.loop(0, NWIN)
        def _(w):
            off = base + w * W
            pltpu.sync_copy(src_hbm.at[pl.ds(off, W)], sidx)
            pltpu.sync_copy(dst_hbm.at[pl.ds(off, W)], didx)
            pltpu.async_copy(h_hbm.at[sidx], rows, sem).wait()
            pltpu.sync_copy(rows, acc_sh.at[didx], add=True)

        plsc.subcore_barrier()
        pltpu.sync_copy(acc_sh.at[pl.ds(row0, RCH)], out_hbm.at[c, pl.ds(row0, RCH)])

    return k(h, src, dst)


_R = 2000


def _prep(degp, features):
    def body(degp_ref, f_ref, ns_ref, nd_ref, h1n_ref):
        dsrc = degp_ref[0, 0] + degp_ref[1, 0]
        ddst = degp_ref[0, 1] + degp_ref[1, 1]
        ns = jnp.where(dsrc > 0, lax.rsqrt(jnp.maximum(dsrc, 1.0)), 0.0)
        nd = jnp.where(ddst > 0, lax.rsqrt(jnp.maximum(ddst, 1.0)), 0.0)
        ns_ref[...] = ns
        nd_ref[...] = nd
        h1n_ref[...] = f_ref[...] * ns

    return pl.pallas_call(
        body,
        grid=(N // _R,),
        in_specs=[
            pl.BlockSpec((NC, 2, _R, 1), lambda i: (0, 0, i, 0)),
            pl.BlockSpec((_R, D), lambda i: (i, 0)),
        ],
        out_specs=[
            pl.BlockSpec((_R, 1), lambda i: (i, 0)),
            pl.BlockSpec((_R, 1), lambda i: (i, 0)),
            pl.BlockSpec((_R, D), lambda i: (i, 0)),
        ],
        out_shape=[
            jax.ShapeDtypeStruct((N, 1), jnp.float32),
            jax.ShapeDtypeStruct((N, 1), jnp.float32),
            jax.ShapeDtypeStruct((N, D), jnp.float32),
        ],
    )(degp.reshape(NC, 2, NP, 1), features)


def _dense(p, nd, Wm, b, ns=None, relu=False):
    def body(*refs):
        if ns is not None:
            p_ref, nd_ref, w_ref, b_ref, ns_ref, o_ref = refs
        else:
            p_ref, nd_ref, w_ref, b_ref, o_ref = refs
        agg = (p_ref[0] + p_ref[1]) * nd_ref[...]
        h = jnp.dot(agg, w_ref[...], preferred_element_type=jnp.float32)
        h = h + b_ref[...]
        if relu:
            h = jnp.maximum(h, 0.0)
        if ns is not None:
            h = h * ns_ref[...]
        o_ref[...] = h

    in_specs = [
        pl.BlockSpec((NC, _R, D), lambda i: (0, i, 0)),
        pl.BlockSpec((_R, 1), lambda i: (i, 0)),
        pl.BlockSpec((D, D), lambda i: (0, 0)),
        pl.BlockSpec((1, D), lambda i: (0, 0)),
    ]
    args = [p, nd, Wm, b.reshape(1, D)]
    if ns is not None:
        in_specs.append(pl.BlockSpec((_R, 1), lambda i: (i, 0)))
        args.append(ns)

    return pl.pallas_call(
        body,
        grid=(N // _R,),
        in_specs=in_specs,
        out_specs=pl.BlockSpec((_R, D), lambda i: (i, 0)),
        out_shape=jax.ShapeDtypeStruct((N, D), jnp.float32),
    )(*args)


def kernel(features, edge_index, W1, b1, W2, b2):
    src = edge_index[0].astype(jnp.int32)
    dst = edge_index[1].astype(jnp.int32)
    degp = _degrees(src, dst)
    ns, nd, h1n = _prep(degp, features)
    p1 = _edge_pass(h1n, src, dst)
    h2n = _dense(p1, nd, W1, b1, ns=ns, relu=True)
    p2 = _edge_pass(h2n, src, dst)
    return _dense(p2, nd, W2, b2)
